# baseline (speedup 1.0000x reference)
import jax
import jax.numpy as jnp
from jax.experimental import pallas as pl

_N = 1048576
_BLOCK_N = 262144


def _add_mask_kernel(x_ref, o_ref):
    j = pl.program_id(0)

    @pl.when(j < 3)
    def _():
        o_ref[...] = x_ref[...] + 1.0

    @pl.when(j >= 3)
    def _():
        o_ref[...] = x_ref[...]


def kernel(x):
    xt = jnp.transpose(x, (2, 1, 0))
    out_t = pl.pallas_call(
        _add_mask_kernel,
        out_shape=jax.ShapeDtypeStruct((5, 3, _N), jnp.float32),
        grid=(5, _N // _BLOCK_N),
        in_specs=[pl.BlockSpec((1, 3, _BLOCK_N), lambda j, k: (j, 0, k))],
        out_specs=pl.BlockSpec((1, 3, _BLOCK_N), lambda j, k: (j, 0, k)),
    )(xt)
    return jnp.transpose(out_t, (2, 1, 0))


# speedup vs baseline: 1.0320x; 1.0320x over previous
import jax
import jax.numpy as jnp
from jax.experimental import pallas as pl

_N = 1048576
_BLOCK_N = 524288


def _add_mask_kernel(x_ref, o_ref):
    j = pl.program_id(0)
    inc = jnp.where(j < 3, 1.0, 0.0).astype(jnp.float32)
    o_ref[...] = x_ref[...] + inc


def kernel(x):
    xt = jnp.transpose(x, (2, 1, 0))
    out_t = pl.pallas_call(
        _add_mask_kernel,
        out_shape=jax.ShapeDtypeStruct((5, 3, _N), jnp.float32),
        grid=(5, _N // _BLOCK_N),
        in_specs=[pl.BlockSpec((1, 3, _BLOCK_N), lambda j, k: (j, 0, k))],
        out_specs=pl.BlockSpec((1, 3, _BLOCK_N), lambda j, k: (j, 0, k)),
    )(xt)
    return jnp.transpose(out_t, (2, 1, 0))
